# Initial kernel scaffold; baseline (speedup 1.0000x reference)
import jax
import jax.numpy as jnp
from jax.experimental import pallas as pl


def kernel(charIDx, edge_index, emb, W1, b1, W2, b2, Wc, bc):
    raise NotImplementedError("write your pallas kernel here")


# speedup vs baseline: 3.1268x; 3.1268x over previous
import functools

import jax
import jax.numpy as jnp
from jax import lax
from jax.experimental import pallas as pl
from jax.experimental.pallas import tpu as pltpu
from jax.experimental.pallas import tpu_sc as plsc

N = 10000
E = 320000
V = 10000
D = 128
H = 128
C = 16

NW = 32
CPW = D // NW
CH = 4000
NCHUNK = E // CH
GRP = CH // 16
UNROLL = 5

_mesh = plsc.VectorSubcoreMesh(core_axis_name="c", subcore_axis_name="s")
_sc_params = pltpu.CompilerParams(needs_layout_passes=False)


def _edge_compute(srcb, dstb, feat, acc, cnt, char, slot, nfeat_rows):
    base = slot * CH
    ones = jnp.full((16,), 1.0, dtype=jnp.float32)

    def group_body(g0, carry):
        off = g0 * (16 * UNROLL)
        for u in range(UNROLL):
            start = off + (base + u * 16)
            s = srcb[pl.ds(start, 16)]
            d = dstb[pl.ds(start, 16)]
            if char is not None:
                idx0 = plsc.load_gather(char, [s])
            else:
                idx0 = s
            for c in range(CPW):
                src_idx = idx0 if c == 0 else idx0 + (c * nfeat_rows)
                v = plsc.load_gather(feat, [src_idx])
                dst_idx = d if c == 0 else d + (c * N)
                plsc.addupdate_scatter(acc, [dst_idx], v)
            if cnt is not None:
                plsc.addupdate_scatter(cnt, [d], ones)
        return carry

    lax.fori_loop(0, GRP // UNROLL, group_body, 0)


def _memset_zero(ref, nwords):
    z = jnp.zeros((16,), dtype=jnp.float32)

    def body(i, carry):
        ref[pl.ds(i * 16, 16)] = z
        return carry

    lax.fori_loop(0, nwords // 16, body, 0)


def _edge_loop(edges, srcb, dstb, ssem0, ssem1, dsem0, dsem1, compute):
    def dma_pair(k, slot, ssem, dsem):
        off = k * CH
        sc = pltpu.make_async_copy(
            edges.at[pl.ds(off, CH)], srcb.at[pl.ds(slot * CH, CH)], ssem)
        dc = pltpu.make_async_copy(
            edges.at[pl.ds(E + off, CH)], dstb.at[pl.ds(slot * CH, CH)], dsem)
        return sc, dc

    def start(k, slot, ssem, dsem):
        sc, dc = dma_pair(k, slot, ssem, dsem)
        sc.start()
        dc.start()

    def wait(k, slot, ssem, dsem):
        sc, dc = dma_pair(k, slot, ssem, dsem)
        sc.wait()
        dc.wait()

    start(0, 0, ssem0, dsem0)
    start(1, 1, ssem1, dsem1)

    def outer(i, carry):
        k0 = 2 * i
        wait(k0, 0, ssem0, dsem0)
        compute(0)
        start(k0 + 2, 0, ssem0, dsem0)
        wait(k0 + 1, 1, ssem1, dsem1)
        compute(1)
        start(k0 + 3, 1, ssem1, dsem1)
        return carry

    lax.fori_loop(0, NCHUNK // 2 - 1, outer, 0)
    wait(NCHUNK - 2, 0, ssem0, dsem0)
    compute(0)
    wait(NCHUNK - 1, 1, ssem1, dsem1)
    compute(1)


@functools.partial(
    pl.kernel,
    mesh=_mesh,
    compiler_params=_sc_params,
    out_type=(
        jax.ShapeDtypeStruct((D * N,), jnp.float32),
        jax.ShapeDtypeStruct((N,), jnp.float32),
    ),
    scratch_types=[
        pltpu.VMEM((CPW * V,), jnp.float32),
        pltpu.VMEM((CPW * N,), jnp.float32),
        pltpu.VMEM((N,), jnp.int32),
        pltpu.VMEM((N,), jnp.float32),
        pltpu.VMEM((2 * CH,), jnp.int32),
        pltpu.VMEM((2 * CH,), jnp.int32),
        pltpu.SemaphoreType.DMA,
        pltpu.SemaphoreType.DMA,
        pltpu.SemaphoreType.DMA,
        pltpu.SemaphoreType.DMA,
    ],
)
def _sc_layer1(embT, edges, charIDx, accT_out, cnt_out,
               feat, acc, char, cnt, srcb, dstb, ssem0, ssem1, dsem0, dsem1):
    wid = lax.axis_index("s") * 2 + lax.axis_index("c")
    c0 = wid * CPW
    pltpu.sync_copy(embT.at[pl.ds(c0 * V, CPW * V)], feat)
    pltpu.sync_copy(charIDx, char)
    _memset_zero(acc, CPW * N)
    _memset_zero(cnt, N)

    def compute(slot):
        _edge_compute(srcb, dstb, feat, acc, cnt, char, slot, V)

    _edge_loop(edges, srcb, dstb, ssem0, ssem1, dsem0, dsem1, compute)

    pltpu.sync_copy(acc, accT_out.at[pl.ds(c0 * N, CPW * N)])

    @pl.when(wid == 0)
    def _():
        pltpu.sync_copy(cnt, cnt_out)


@functools.partial(
    pl.kernel,
    mesh=_mesh,
    compiler_params=_sc_params,
    out_type=jax.ShapeDtypeStruct((D * N,), jnp.float32),
    scratch_types=[
        pltpu.VMEM((CPW * N,), jnp.float32),
        pltpu.VMEM((CPW * N,), jnp.float32),
        pltpu.VMEM((2 * CH,), jnp.int32),
        pltpu.VMEM((2 * CH,), jnp.int32),
        pltpu.SemaphoreType.DMA,
        pltpu.SemaphoreType.DMA,
        pltpu.SemaphoreType.DMA,
        pltpu.SemaphoreType.DMA,
    ],
)
def _sc_layer2(hT, edges, accT_out,
               feat, acc, srcb, dstb, ssem0, ssem1, dsem0, dsem1):
    wid = lax.axis_index("s") * 2 + lax.axis_index("c")
    c0 = wid * CPW
    pltpu.sync_copy(hT.at[pl.ds(c0 * N, CPW * N)], feat)
    _memset_zero(acc, CPW * N)

    def compute(slot):
        _edge_compute(srcb, dstb, feat, acc, None, None, slot, N)

    _edge_loop(edges, srcb, dstb, ssem0, ssem1, dsem0, dsem1, compute)

    pltpu.sync_copy(acc, accT_out.at[pl.ds(c0 * N, CPW * N)])


def _tc_layer_body(w_ref, acc_ref, cnt_ref, b_ref, out_ref):
    y = lax.dot_general(w_ref[...], acc_ref[...],
                        (((1,), (0,)), ((), ())),
                        preferred_element_type=jnp.float32)
    inv = 1.0 / jnp.maximum(cnt_ref[...], 1.0)
    out_ref[...] = jnp.maximum(y * inv + b_ref[...], 0.0)


def _tc_head_body(w_ref, acc_ref, cnt_ref, b_ref, wc_ref, bc_ref, out_ref):
    y = lax.dot_general(w_ref[...], acc_ref[...],
                        (((1,), (0,)), ((), ())),
                        preferred_element_type=jnp.float32)
    inv = 1.0 / jnp.maximum(cnt_ref[...], 1.0)
    h = jnp.maximum(y * inv + b_ref[...], 0.0)
    hg = jnp.sum(h, axis=1, keepdims=True) * (1.0 / N)
    out_ref[...] = lax.dot_general(wc_ref[...], hg,
                                   (((1,), (0,)), ((), ())),
                                   preferred_element_type=jnp.float32) + bc_ref[...]


def kernel(charIDx, edge_index, emb, W1, b1, W2, b2, Wc, bc):
    edges = edge_index.reshape(-1).astype(jnp.int32)
    embT_flat = emb.T.reshape(-1)

    acc1_flat, cnt = _sc_layer1(embT_flat, edges, charIDx.astype(jnp.int32))
    acc1 = acc1_flat.reshape(D, N)
    cnt_row = cnt.reshape(1, N)

    h1T = pl.pallas_call(
        _tc_layer_body,
        out_shape=jax.ShapeDtypeStruct((H, N), jnp.float32),
    )(W1, acc1, cnt_row, b1.reshape(H, 1))

    acc2_flat = _sc_layer2(h1T.reshape(-1), edges)
    acc2 = acc2_flat.reshape(H, N)

    out = pl.pallas_call(
        _tc_head_body,
        out_shape=jax.ShapeDtypeStruct((C, 1), jnp.float32),
    )(W2, acc2, cnt_row, b2.reshape(H, 1), Wc, bc.reshape(C, 1))

    return out.reshape(1, C)


# speedup vs baseline: 7.7389x; 2.4750x over previous
import functools

import jax
import jax.numpy as jnp
from jax import lax
from jax.experimental import pallas as pl
from jax.experimental.pallas import tpu as pltpu
from jax.experimental.pallas import tpu_sc as plsc

N = 10000
E = 320000
V = 10000
D = 128
H = 128
C = 16

NW = 32
CPW = D // NW
CH = 4000
NCHUNK = E // CH
GRP = CH // 16
UNROLL = 5

_mesh = plsc.VectorSubcoreMesh(core_axis_name="c", subcore_axis_name="s")
_sc_params = pltpu.CompilerParams(needs_layout_passes=False)


def _edge_compute(srcb, dstb, feat, acc, cnt, char, slot, nfeat_rows):
    base = slot * CH
    ones = jnp.full((16,), 1.0, dtype=jnp.float32)

    @plsc.parallel_loop(0, GRP, 1, unroll=UNROLL)
    def _(g):
        start = g * 16 + base
        s = srcb[pl.ds(start, 16)]
        d = dstb[pl.ds(start, 16)]
        if char is not None:
            idx0 = plsc.load_gather(char, [s])
        else:
            idx0 = s
        for c in range(CPW):
            src_idx = idx0 if c == 0 else idx0 + (c * nfeat_rows)
            v = plsc.load_gather(feat, [src_idx])
            dst_idx = d if c == 0 else d + (c * N)
            plsc.addupdate_scatter(acc, [dst_idx], v)
        if cnt is not None:
            plsc.addupdate_scatter(cnt, [d], ones)


def _memset_zero(ref, nwords):
    z = jnp.zeros((16,), dtype=jnp.float32)

    @plsc.parallel_loop(0, nwords // 16, 1, unroll=8)
    def _(i):
        ref[pl.ds(i * 16, 16)] = z


def _edge_loop(edges, srcb, dstb, ssem0, ssem1, dsem0, dsem1, compute):
    def dma_pair(k, slot, ssem, dsem):
        off = k * CH
        sc = pltpu.make_async_copy(
            edges.at[pl.ds(off, CH)], srcb.at[pl.ds(slot * CH, CH)], ssem)
        dc = pltpu.make_async_copy(
            edges.at[pl.ds(E + off, CH)], dstb.at[pl.ds(slot * CH, CH)], dsem)
        return sc, dc

    def start(k, slot, ssem, dsem):
        sc, dc = dma_pair(k, slot, ssem, dsem)
        sc.start()
        dc.start()

    def wait(k, slot, ssem, dsem):
        sc, dc = dma_pair(k, slot, ssem, dsem)
        sc.wait()
        dc.wait()

    start(0, 0, ssem0, dsem0)
    start(1, 1, ssem1, dsem1)

    def outer(i, carry):
        k0 = 2 * i
        wait(k0, 0, ssem0, dsem0)
        compute(0)
        start(k0 + 2, 0, ssem0, dsem0)
        wait(k0 + 1, 1, ssem1, dsem1)
        compute(1)
        start(k0 + 3, 1, ssem1, dsem1)
        return carry

    lax.fori_loop(0, NCHUNK // 2 - 1, outer, 0)
    wait(NCHUNK - 2, 0, ssem0, dsem0)
    compute(0)
    wait(NCHUNK - 1, 1, ssem1, dsem1)
    compute(1)


@functools.partial(
    pl.kernel,
    mesh=_mesh,
    compiler_params=_sc_params,
    out_type=(
        jax.ShapeDtypeStruct((D * N,), jnp.float32),
        jax.ShapeDtypeStruct((N,), jnp.float32),
    ),
    scratch_types=[
        pltpu.VMEM((CPW * V,), jnp.float32),
        pltpu.VMEM((CPW * N,), jnp.float32),
        pltpu.VMEM((N,), jnp.int32),
        pltpu.VMEM((N,), jnp.float32),
        pltpu.VMEM((2 * CH,), jnp.int32),
        pltpu.VMEM((2 * CH,), jnp.int32),
        pltpu.SemaphoreType.DMA,
        pltpu.SemaphoreType.DMA,
        pltpu.SemaphoreType.DMA,
        pltpu.SemaphoreType.DMA,
    ],
)
def _sc_layer1(embT, edges, charIDx, accT_out, cnt_out,
               feat, acc, char, cnt, srcb, dstb, ssem0, ssem1, dsem0, dsem1):
    wid = lax.axis_index("s") * 2 + lax.axis_index("c")
    c0 = wid * CPW
    pltpu.sync_copy(embT.at[pl.ds(c0 * V, CPW * V)], feat)
    pltpu.sync_copy(charIDx, char)
    _memset_zero(acc, CPW * N)
    _memset_zero(cnt, N)

    def compute(slot):
        _edge_compute(srcb, dstb, feat, acc, cnt, char, slot, V)

    _edge_loop(edges, srcb, dstb, ssem0, ssem1, dsem0, dsem1, compute)

    pltpu.sync_copy(acc, accT_out.at[pl.ds(c0 * N, CPW * N)])

    @pl.when(wid == 0)
    def _():
        pltpu.sync_copy(cnt, cnt_out)


@functools.partial(
    pl.kernel,
    mesh=_mesh,
    compiler_params=_sc_params,
    out_type=jax.ShapeDtypeStruct((D * N,), jnp.float32),
    scratch_types=[
        pltpu.VMEM((CPW * N,), jnp.float32),
        pltpu.VMEM((CPW * N,), jnp.float32),
        pltpu.VMEM((2 * CH,), jnp.int32),
        pltpu.VMEM((2 * CH,), jnp.int32),
        pltpu.SemaphoreType.DMA,
        pltpu.SemaphoreType.DMA,
        pltpu.SemaphoreType.DMA,
        pltpu.SemaphoreType.DMA,
    ],
)
def _sc_layer2(hT, edges, accT_out,
               feat, acc, srcb, dstb, ssem0, ssem1, dsem0, dsem1):
    wid = lax.axis_index("s") * 2 + lax.axis_index("c")
    c0 = wid * CPW
    pltpu.sync_copy(hT.at[pl.ds(c0 * N, CPW * N)], feat)
    _memset_zero(acc, CPW * N)

    def compute(slot):
        _edge_compute(srcb, dstb, feat, acc, None, None, slot, N)

    _edge_loop(edges, srcb, dstb, ssem0, ssem1, dsem0, dsem1, compute)

    pltpu.sync_copy(acc, accT_out.at[pl.ds(c0 * N, CPW * N)])


def _tc_layer_body(w_ref, acc_ref, cnt_ref, b_ref, out_ref):
    y = lax.dot_general(w_ref[...], acc_ref[...],
                        (((1,), (0,)), ((), ())),
                        preferred_element_type=jnp.float32)
    inv = 1.0 / jnp.maximum(cnt_ref[...], 1.0)
    out_ref[...] = jnp.maximum(y * inv + b_ref[...], 0.0)


def _tc_head_body(w_ref, acc_ref, cnt_ref, b_ref, wc_ref, bc_ref, out_ref):
    y = lax.dot_general(w_ref[...], acc_ref[...],
                        (((1,), (0,)), ((), ())),
                        preferred_element_type=jnp.float32)
    inv = 1.0 / jnp.maximum(cnt_ref[...], 1.0)
    h = jnp.maximum(y * inv + b_ref[...], 0.0)
    hg = jnp.sum(h, axis=1, keepdims=True) * (1.0 / N)
    out_ref[...] = lax.dot_general(wc_ref[...], hg,
                                   (((1,), (0,)), ((), ())),
                                   preferred_element_type=jnp.float32) + bc_ref[...]


def kernel(charIDx, edge_index, emb, W1, b1, W2, b2, Wc, bc):
    edges = edge_index.reshape(-1).astype(jnp.int32)
    embT_flat = emb.T.reshape(-1)

    acc1_flat, cnt = _sc_layer1(embT_flat, edges, charIDx.astype(jnp.int32))
    acc1 = acc1_flat.reshape(D, N)
    cnt_row = cnt.reshape(1, N)

    h1T = pl.pallas_call(
        _tc_layer_body,
        out_shape=jax.ShapeDtypeStruct((H, N), jnp.float32),
    )(W1, acc1, cnt_row, b1.reshape(H, 1))

    acc2_flat = _sc_layer2(h1T.reshape(-1), edges)
    acc2 = acc2_flat.reshape(H, N)

    out = pl.pallas_call(
        _tc_head_body,
        out_shape=jax.ShapeDtypeStruct((C, 1), jnp.float32),
    )(W2, acc2, cnt_row, b2.reshape(H, 1), Wc, bc.reshape(C, 1))

    return out.reshape(1, C)
